# baseline (speedup 1.0000x reference)
import jax
import jax.numpy as jnp
from jax.experimental import pallas as pl
from jax.experimental.pallas import tpu as pltpu

NUM_EXPERTS = 16
TOP_K = 2

CHUNKS = (1024, 1024, 1024, 512, 256, 128, 64, 64)
NBUF = 3
MAXC = max(CHUNKS)


def _router_manual(x_ref, w_ref, b_ref, o_ref, xbuf, sems):
    starts = []
    s = 0
    for ct in CHUNKS:
        starts.append(s)
        s += ct
    nc = len(CHUNKS)

    def copy(c):
        ct = CHUNKS[c]
        slot = c % NBUF
        return pltpu.make_async_copy(
            x_ref.at[pl.ds(starts[c], ct), :],
            xbuf.at[slot, pl.ds(0, ct), :],
            sems.at[slot],
        )

    for c in range(min(NBUF, nc)):
        copy(c).start()

    w = w_ref[...]
    bias = b_ref[...]
    for c in range(nc):
        copy(c).wait()
        ct = CHUNKS[c]
        st = starts[c]
        xc = xbuf[c % NBUF, 0:ct, :]
        logits = jnp.dot(xc, w, preferred_element_type=jnp.float32) + bias

        m = jnp.max(logits, axis=-1, keepdims=True)
        e = jnp.exp(logits - m)
        p = e * (1.0 / jnp.sum(e, axis=-1, keepdims=True))

        ii = jax.lax.broadcasted_iota(jnp.int32, logits.shape, 1)
        i1 = jnp.argmax(logits, axis=-1, keepdims=True)
        sel1 = ii == i1
        i2 = jnp.argmax(jnp.where(sel1, -jnp.inf, logits), axis=-1, keepdims=True)
        mask = sel1 | (ii == i2)
        o_ref[st:st + ct, :] = jnp.where(mask, p, 0.0)

        if c + NBUF < nc:
            copy(c + NBUF).start()


def kernel(token_inputs, W, b, num_experts):
    B, S, D = token_inputs.shape
    E = W.shape[1]
    x = token_inputs.reshape(B * S, D)
    b2 = b.reshape(1, E)
    out = pl.pallas_call(
        _router_manual,
        in_specs=[
            pl.BlockSpec(memory_space=pltpu.MemorySpace.HBM),
            pl.BlockSpec((D, E), lambda: (0, 0)),
            pl.BlockSpec((1, E), lambda: (0, 0)),
        ],
        out_specs=pl.BlockSpec((B * S, E), lambda: (0, 0)),
        out_shape=jax.ShapeDtypeStruct((B * S, E), jnp.float32),
        scratch_shapes=[
            pltpu.VMEM((NBUF, MAXC, D), jnp.float32),
            pltpu.SemaphoreType.DMA((NBUF,)),
        ],
    )(x, W, b2)
    return out.reshape(B, S, E)
